# baseline (speedup 1.0000x reference)
import functools

import jax
import jax.numpy as jnp
from jax import lax
from jax.experimental import pallas as pl
from jax.experimental.pallas import tpu as pltpu
from jax.experimental.pallas import tpu_sc as plsc

D = 1024
H = 2048
E = 8
FLAGS_DIM = 8
T = 2048
BLOCK = 128

CAP = 2048
NSLOT = E * CAP
G = 128
TPG = CAP // G
NTILE = (2 * T) // G + E
NW = 32
TW = T // NW
GREP = 128


def _router_body(x_ref, f_ref, rwx_ref, rwe_ref, rwf_ref, rb_ref,
                 pinv_ref, pscale_ref, pcarry_ref,
                 didx_ref, gate_ref, counts_ref, carry_ref, run_ref):
    i = pl.program_id(0)

    @pl.when(i == 0)
    def _():
        carry_ref[...] = jnp.zeros_like(carry_ref)
        run_ref[...] = jnp.zeros_like(run_ref)

    xb = x_ref[...]
    u = xb * pinv_ref[:, :1]
    p = u
    for k in (1, 2, 4, 8, 16, 32, 64):
        shifted = jnp.concatenate(
            [jnp.zeros((k, D), jnp.float32), p[:BLOCK - k, :]], axis=0)
        p = p + shifted
    yb = pcarry_ref[:, :1] * carry_ref[0:1, :] + pscale_ref[:, :1] * p
    yb = jnp.where(jnp.isfinite(yb), yb, 0.0)
    carry_ref[0:1, :] = yb[BLOCK - 1:BLOCK, :]

    fb = f_ref[...]
    logits = (
        jax.lax.dot_general(xb, rwx_ref[...], (((1,), (0,)), ((), ())),
                            preferred_element_type=jnp.float32)
        + jax.lax.dot_general(yb, rwe_ref[...], (((1,), (0,)), ((), ())),
                              preferred_element_type=jnp.float32)
        + jax.lax.dot_general(fb, rwf_ref[...], (((1,), (0,)), ((), ())),
                              preferred_element_type=jnp.float32)
        + rb_ref[0:1, :]
    )

    lane = jax.lax.broadcasted_iota(jnp.int32, (BLOCK, E), 1)
    m1 = jnp.max(logits, axis=1, keepdims=True)
    idx1 = jnp.min(jnp.where(logits == m1, lane, E), axis=1, keepdims=True)
    sel1 = lane == idx1
    masked = jnp.where(sel1, -jnp.inf, logits)
    m2 = jnp.max(masked, axis=1, keepdims=True)
    idx2 = jnp.min(jnp.where(masked == m2, lane, E), axis=1, keepdims=True)
    sel2 = lane == idx2
    ex = jnp.exp(m2 - m1)
    s = 1.0 + ex
    g1 = 1.0 / s
    g2 = ex / s

    blk = sel1.astype(jnp.int32) + sel2.astype(jnp.int32)
    pc = blk
    for k in (1, 2, 4, 8, 16, 32, 64):
        shifted = jnp.concatenate(
            [jnp.zeros((k, E), jnp.int32), pc[:BLOCK - k, :]], axis=0)
        pc = pc + shifted
    excl = pc - blk
    runrow = run_ref[0:1, :]
    base = excl + runrow + lane * CAP
    d0 = jnp.sum(jnp.where(sel1, base, 0), axis=1, keepdims=True)
    d1 = jnp.sum(jnp.where(sel2, base, 0), axis=1, keepdims=True)
    run_ref[0:1, :] = runrow + pc[BLOCK - 1:BLOCK, :]
    counts_ref[...] = jnp.broadcast_to(run_ref[0:1, :], (8, E))

    didx_ref[...] = jnp.where(lane == 0, d0, 0) + jnp.where(lane == 1, d1, 0)
    gate_ref[...] = jnp.where(lane == 0, g1, 0.0) + jnp.where(lane == 1, g2, 0.0)


_vector_mesh = plsc.VectorSubcoreMesh(core_axis_name="c", subcore_axis_name="s")


@functools.partial(
    pl.kernel,
    out_type=[jax.ShapeDtypeStruct((NSLOT, D), jnp.float32),
              jax.ShapeDtypeStruct((NSLOT, GREP), jnp.float32)],
    mesh=_vector_mesh,
    scratch_types=[pltpu.VMEM((TW,), jnp.int32),
                   pltpu.VMEM((TW,), jnp.int32),
                   pltpu.VMEM((TW, D), jnp.float32),
                   pltpu.VMEM((TW, GREP), jnp.float32)],
)
def _sc_dispatch(x_hbm, d0_hbm, d1_hbm, g0_hbm, g1_hbm,
                 xs_hbm, gs_hbm, i0_v, i1_v, xrows_v, grows_v):
    wid = lax.axis_index("s") * 2 + lax.axis_index("c")
    base = wid * TW
    pltpu.sync_copy(d0_hbm.at[wid], i0_v)
    pltpu.sync_copy(d1_hbm.at[wid], i1_v)
    pltpu.sync_copy(x_hbm.at[pl.ds(base, TW)], xrows_v)
    pltpu.sync_copy(xrows_v, xs_hbm.at[i0_v])
    pltpu.sync_copy(xrows_v, xs_hbm.at[i1_v])
    pltpu.sync_copy(g0_hbm.at[pl.ds(base, TW)], grows_v)
    pltpu.sync_copy(grows_v, gs_hbm.at[i0_v])
    pltpu.sync_copy(g1_hbm.at[pl.ds(base, TW)], grows_v)
    pltpu.sync_copy(grows_v, gs_hbm.at[i1_v])


def _gmm_body(te_ref, off_ref, xs_ref, gs_ref, w1_ref, w2_ref, ys_ref):
    h = jax.lax.dot_general(xs_ref[...], w1_ref[0], (((1,), (1,)), ((), ())),
                            preferred_element_type=jnp.float32)
    h = jnp.maximum(h, 0.0)
    y = jax.lax.dot_general(h, w2_ref[0], (((1,), (1,)), ((), ())),
                            preferred_element_type=jnp.float32)
    ys_ref[...] = y * gs_ref[:, 0:1]


CW = TW // 2


@functools.partial(
    pl.kernel,
    out_type=jax.ShapeDtypeStruct((T, D), jnp.float32),
    mesh=_vector_mesh,
    scratch_types=[pltpu.VMEM((CW,), jnp.int32),
                   pltpu.VMEM((CW,), jnp.int32),
                   pltpu.VMEM((CW, D), jnp.float32),
                   pltpu.VMEM((CW, D), jnp.float32)],
)
def _sc_combine(ys_hbm, d0_hbm, d1_hbm, out_hbm, i0_v, i1_v, b0_v, b1_v):
    wid = lax.axis_index("s") * 2 + lax.axis_index("c")
    for ci in range(2):
        row = wid * 2 + ci
        pltpu.sync_copy(d0_hbm.at[row], i0_v)
        pltpu.sync_copy(d1_hbm.at[row], i1_v)
        pltpu.sync_copy(ys_hbm.at[i0_v], b0_v)
        pltpu.sync_copy(ys_hbm.at[i1_v], b1_v)

        @pl.loop(0, CW)
        def _(r):
            for j in range(D // 16):
                sl = (r, pl.ds(j * 16, 16))
                b0_v[sl] = b0_v[sl] + b1_v[sl]

        pltpu.sync_copy(b0_v, out_hbm.at[pl.ds(row * CW, CW)])


def kernel(x, flags, W1, W2, router_w, router_b, alpha, layer_idx):
    x2 = x.reshape(T, D)
    flags2 = flags.reshape(T, FLAGS_DIM)
    rw = jax.lax.dynamic_index_in_dim(router_w, layer_idx, 0, keepdims=False)
    rb = jax.lax.dynamic_index_in_dim(router_b, layer_idx, 0, keepdims=False)
    a = jnp.clip(
        jnp.nan_to_num(alpha[layer_idx].astype(jnp.float32),
                       nan=0.8, posinf=0.8, neginf=0.8),
        1e-4, 0.9999)
    ar = jnp.arange(BLOCK, dtype=jnp.float32)
    pow_a = a ** ar
    pow_ap1 = a ** (ar + 1.0)
    pow_inv = a ** (-ar)
    one_minus = 1.0 - a
    P_inv = jnp.broadcast_to(pow_inv[:, None], (BLOCK, BLOCK))
    P_scale = jnp.broadcast_to((one_minus * pow_a)[:, None], (BLOCK, BLOCK))
    P_carry = jnp.broadcast_to(pow_ap1[:, None], (BLOCK, BLOCK))
    rwxT = rw[:, :D].T
    rweT = rw[:, D:2 * D].T
    rwfT = rw[:, 2 * D:].T
    rb8 = jnp.broadcast_to(rb[None, :], (8, E))

    didx, gate, counts8 = pl.pallas_call(
        _router_body,
        grid=(T // BLOCK,),
        in_specs=[
            pl.BlockSpec((BLOCK, D), lambda i: (i, 0)),
            pl.BlockSpec((BLOCK, FLAGS_DIM), lambda i: (i, 0)),
            pl.BlockSpec((D, E), lambda i: (0, 0)),
            pl.BlockSpec((D, E), lambda i: (0, 0)),
            pl.BlockSpec((FLAGS_DIM, E), lambda i: (0, 0)),
            pl.BlockSpec((8, E), lambda i: (0, 0)),
            pl.BlockSpec((BLOCK, BLOCK), lambda i: (0, 0)),
            pl.BlockSpec((BLOCK, BLOCK), lambda i: (0, 0)),
            pl.BlockSpec((BLOCK, BLOCK), lambda i: (0, 0)),
        ],
        out_specs=[
            pl.BlockSpec((BLOCK, E), lambda i: (i, 0)),
            pl.BlockSpec((BLOCK, E), lambda i: (i, 0)),
            pl.BlockSpec((8, E), lambda i: (0, 0)),
        ],
        out_shape=[
            jax.ShapeDtypeStruct((T, E), jnp.int32),
            jax.ShapeDtypeStruct((T, E), jnp.float32),
            jax.ShapeDtypeStruct((8, E), jnp.int32),
        ],
        scratch_shapes=[pltpu.VMEM((8, D), jnp.float32),
                        pltpu.VMEM((8, E), jnp.int32)],
    )(x2, flags2, rwxT, rweT, rwfT, rb8, P_inv, P_scale, P_carry)

    counts = counts8[0]
    nt = (counts + (G - 1)) // G
    cum = jnp.cumsum(nt)
    i40 = jnp.arange(NTILE, dtype=jnp.int32)
    te = jnp.sum((i40[:, None] >= cum[None, :]).astype(jnp.int32), axis=1)
    overflow = te >= E
    te = jnp.where(overflow, E - 1, te).astype(jnp.int32)
    cum_excl = cum - nt
    off = jnp.where(overflow, jnp.maximum(nt[E - 1] - 1, 0),
                    i40 - cum_excl[te]).astype(jnp.int32)

    d0 = didx[:, 0]
    d1 = didx[:, 1]
    g0 = gate[:, 0]
    g1 = gate[:, 1]
    d0w = d0.reshape(NW, TW)
    d1w = d1.reshape(NW, TW)
    d0c = d0.reshape(NW * 2, CW)
    d1c = d1.reshape(NW * 2, CW)
    g0rep = jnp.broadcast_to(g0[:, None], (T, GREP))
    g1rep = jnp.broadcast_to(g1[:, None], (T, GREP))

    xs, gs = _sc_dispatch(x2, d0w, d1w, g0rep, g1rep)

    grid_spec = pltpu.PrefetchScalarGridSpec(
        num_scalar_prefetch=2,
        grid=(NTILE,),
        in_specs=[
            pl.BlockSpec((G, D), lambda i, t, o: (t[i] * TPG + o[i], 0)),
            pl.BlockSpec((G, GREP), lambda i, t, o: (t[i] * TPG + o[i], 0)),
            pl.BlockSpec((1, H, D), lambda i, t, o: (t[i], 0, 0)),
            pl.BlockSpec((1, D, H), lambda i, t, o: (t[i], 0, 0)),
        ],
        out_specs=pl.BlockSpec((G, D), lambda i, t, o: (t[i] * TPG + o[i], 0)),
    )
    ys = pl.pallas_call(
        _gmm_body,
        grid_spec=grid_spec,
        out_shape=jax.ShapeDtypeStruct((NSLOT, D), jnp.float32),
    )(te, off, xs, gs, W1, W2)

    out2 = _sc_combine(ys, d0c, d1c)
    return out2.reshape(1, T, D)


# speedup vs baseline: 1.3211x; 1.3211x over previous
import functools

import jax
import jax.numpy as jnp
from jax import lax
from jax.experimental import pallas as pl
from jax.experimental.pallas import tpu as pltpu
from jax.experimental.pallas import tpu_sc as plsc

D = 1024
H = 2048
E = 8
FLAGS_DIM = 8
T = 2048
BLOCK = 128

CAP = 2048
NSLOT = E * CAP
G = 256
TPG = CAP // G
NTILE = (2 * T) // G + E
NW = 32
TW = T // NW
GREP = 128


def _router_body(x_ref, f_ref, rwx_ref, rwe_ref, rwf_ref, rb_ref,
                 pinv_ref, pscale_ref, pcarry_ref,
                 didx_ref, gate_ref, counts_ref, carry_ref, run_ref):
    i = pl.program_id(0)

    @pl.when(i == 0)
    def _():
        carry_ref[...] = jnp.zeros_like(carry_ref)
        run_ref[...] = jnp.zeros_like(run_ref)

    xb = x_ref[...]
    u = xb * pinv_ref[:, :1]
    p = u
    for k in (1, 2, 4, 8, 16, 32, 64):
        shifted = jnp.concatenate(
            [jnp.zeros((k, D), jnp.float32), p[:BLOCK - k, :]], axis=0)
        p = p + shifted
    yb = pcarry_ref[:, :1] * carry_ref[0:1, :] + pscale_ref[:, :1] * p
    yb = jnp.where(jnp.isfinite(yb), yb, 0.0)
    carry_ref[0:1, :] = yb[BLOCK - 1:BLOCK, :]

    fb = f_ref[...]
    logits = (
        jax.lax.dot_general(xb, rwx_ref[...], (((1,), (0,)), ((), ())),
                            preferred_element_type=jnp.float32)
        + jax.lax.dot_general(yb, rwe_ref[...], (((1,), (0,)), ((), ())),
                              preferred_element_type=jnp.float32)
        + jax.lax.dot_general(fb, rwf_ref[...], (((1,), (0,)), ((), ())),
                              preferred_element_type=jnp.float32)
        + rb_ref[0:1, :]
    )

    lane = jax.lax.broadcasted_iota(jnp.int32, (BLOCK, E), 1)
    m1 = jnp.max(logits, axis=1, keepdims=True)
    idx1 = jnp.min(jnp.where(logits == m1, lane, E), axis=1, keepdims=True)
    sel1 = lane == idx1
    masked = jnp.where(sel1, -jnp.inf, logits)
    m2 = jnp.max(masked, axis=1, keepdims=True)
    idx2 = jnp.min(jnp.where(masked == m2, lane, E), axis=1, keepdims=True)
    sel2 = lane == idx2
    ex = jnp.exp(m2 - m1)
    s = 1.0 + ex
    g1 = 1.0 / s
    g2 = ex / s

    blk = sel1.astype(jnp.int32) + sel2.astype(jnp.int32)
    pc = blk
    for k in (1, 2, 4, 8, 16, 32, 64):
        shifted = jnp.concatenate(
            [jnp.zeros((k, E), jnp.int32), pc[:BLOCK - k, :]], axis=0)
        pc = pc + shifted
    excl = pc - blk
    runrow = run_ref[0:1, :]
    base = excl + runrow + lane * CAP
    d0 = jnp.sum(jnp.where(sel1, base, 0), axis=1, keepdims=True)
    d1 = jnp.sum(jnp.where(sel2, base, 0), axis=1, keepdims=True)
    run_ref[0:1, :] = runrow + pc[BLOCK - 1:BLOCK, :]
    counts_ref[...] = jnp.broadcast_to(run_ref[0:1, :], (8, E))

    didx_ref[...] = jnp.where(lane == 0, d0, 0) + jnp.where(lane == 1, d1, 0)
    gate_ref[...] = jnp.where(lane == 0, g1, 0.0) + jnp.where(lane == 1, g2, 0.0)


_vector_mesh = plsc.VectorSubcoreMesh(core_axis_name="c", subcore_axis_name="s")


@functools.partial(
    pl.kernel,
    out_type=[jax.ShapeDtypeStruct((NSLOT, D), jnp.float32),
              jax.ShapeDtypeStruct((NSLOT, GREP), jnp.float32)],
    mesh=_vector_mesh,
    scratch_types=[pltpu.VMEM((TW,), jnp.int32),
                   pltpu.VMEM((TW,), jnp.int32),
                   pltpu.VMEM((TW, D), jnp.float32),
                   pltpu.VMEM((TW, GREP), jnp.float32)],
)
def _sc_dispatch(x_hbm, d0_hbm, d1_hbm, g0_hbm, g1_hbm,
                 xs_hbm, gs_hbm, i0_v, i1_v, xrows_v, grows_v):
    wid = lax.axis_index("s") * 2 + lax.axis_index("c")
    base = wid * TW
    pltpu.sync_copy(d0_hbm.at[wid], i0_v)
    pltpu.sync_copy(d1_hbm.at[wid], i1_v)
    pltpu.sync_copy(x_hbm.at[pl.ds(base, TW)], xrows_v)
    pltpu.sync_copy(xrows_v, xs_hbm.at[i0_v])
    pltpu.sync_copy(xrows_v, xs_hbm.at[i1_v])
    pltpu.sync_copy(g0_hbm.at[pl.ds(base, TW)], grows_v)
    pltpu.sync_copy(grows_v, gs_hbm.at[i0_v])
    pltpu.sync_copy(g1_hbm.at[pl.ds(base, TW)], grows_v)
    pltpu.sync_copy(grows_v, gs_hbm.at[i1_v])


def _gmm_body(te_ref, off_ref, valid_ref, xs_ref, gs_ref, w1_ref, w2_ref,
              ys_ref):
    i = pl.program_id(0)

    @pl.when(valid_ref[i] != 0)
    def _():
        h = jax.lax.dot_general(xs_ref[...], w1_ref[0],
                                (((1,), (1,)), ((), ())),
                                preferred_element_type=jnp.float32)
        h = jnp.maximum(h, 0.0)
        y = jax.lax.dot_general(h, w2_ref[0], (((1,), (1,)), ((), ())),
                                preferred_element_type=jnp.float32)
        ys_ref[...] = y * gs_ref[:, 0:1]


CW = TW // 2


@functools.partial(
    pl.kernel,
    out_type=jax.ShapeDtypeStruct((T, D), jnp.float32),
    mesh=_vector_mesh,
    scratch_types=[pltpu.VMEM((CW,), jnp.int32),
                   pltpu.VMEM((CW,), jnp.int32),
                   pltpu.VMEM((CW, D), jnp.float32),
                   pltpu.VMEM((CW, D), jnp.float32)],
)
def _sc_combine(ys_hbm, d0_hbm, d1_hbm, out_hbm, i0_v, i1_v, b0_v, b1_v):
    wid = lax.axis_index("s") * 2 + lax.axis_index("c")
    for ci in range(2):
        row = wid * 2 + ci
        pltpu.sync_copy(d0_hbm.at[row], i0_v)
        pltpu.sync_copy(d1_hbm.at[row], i1_v)
        pltpu.sync_copy(ys_hbm.at[i0_v], b0_v)
        pltpu.sync_copy(ys_hbm.at[i1_v], b1_v)

        @pl.loop(0, CW)
        def _(r):
            for j in range(D // 16):
                sl = (r, pl.ds(j * 16, 16))
                b0_v[sl] = b0_v[sl] + b1_v[sl]

        pltpu.sync_copy(b0_v, out_hbm.at[pl.ds(row * CW, CW)])


def kernel(x, flags, W1, W2, router_w, router_b, alpha, layer_idx):
    x2 = x.reshape(T, D)
    flags2 = flags.reshape(T, FLAGS_DIM)
    rw = jax.lax.dynamic_index_in_dim(router_w, layer_idx, 0, keepdims=False)
    rb = jax.lax.dynamic_index_in_dim(router_b, layer_idx, 0, keepdims=False)
    a = jnp.clip(
        jnp.nan_to_num(alpha[layer_idx].astype(jnp.float32),
                       nan=0.8, posinf=0.8, neginf=0.8),
        1e-4, 0.9999)
    ar = jnp.arange(BLOCK, dtype=jnp.float32)
    pow_a = a ** ar
    pow_ap1 = a ** (ar + 1.0)
    pow_inv = a ** (-ar)
    one_minus = 1.0 - a
    P_inv = jnp.broadcast_to(pow_inv[:, None], (BLOCK, BLOCK))
    P_scale = jnp.broadcast_to((one_minus * pow_a)[:, None], (BLOCK, BLOCK))
    P_carry = jnp.broadcast_to(pow_ap1[:, None], (BLOCK, BLOCK))
    rwxT = rw[:, :D].T
    rweT = rw[:, D:2 * D].T
    rwfT = rw[:, 2 * D:].T
    rb8 = jnp.broadcast_to(rb[None, :], (8, E))

    didx, gate, counts8 = pl.pallas_call(
        _router_body,
        grid=(T // BLOCK,),
        in_specs=[
            pl.BlockSpec((BLOCK, D), lambda i: (i, 0)),
            pl.BlockSpec((BLOCK, FLAGS_DIM), lambda i: (i, 0)),
            pl.BlockSpec((D, E), lambda i: (0, 0)),
            pl.BlockSpec((D, E), lambda i: (0, 0)),
            pl.BlockSpec((FLAGS_DIM, E), lambda i: (0, 0)),
            pl.BlockSpec((8, E), lambda i: (0, 0)),
            pl.BlockSpec((BLOCK, BLOCK), lambda i: (0, 0)),
            pl.BlockSpec((BLOCK, BLOCK), lambda i: (0, 0)),
            pl.BlockSpec((BLOCK, BLOCK), lambda i: (0, 0)),
        ],
        out_specs=[
            pl.BlockSpec((BLOCK, E), lambda i: (i, 0)),
            pl.BlockSpec((BLOCK, E), lambda i: (i, 0)),
            pl.BlockSpec((8, E), lambda i: (0, 0)),
        ],
        out_shape=[
            jax.ShapeDtypeStruct((T, E), jnp.int32),
            jax.ShapeDtypeStruct((T, E), jnp.float32),
            jax.ShapeDtypeStruct((8, E), jnp.int32),
        ],
        scratch_shapes=[pltpu.VMEM((8, D), jnp.float32),
                        pltpu.VMEM((8, E), jnp.int32)],
    )(x2, flags2, rwxT, rweT, rwfT, rb8, P_inv, P_scale, P_carry)

    counts = counts8[0]
    nt = (counts + (G - 1)) // G
    cum = jnp.cumsum(nt)
    i40 = jnp.arange(NTILE, dtype=jnp.int32)
    te = jnp.sum((i40[:, None] >= cum[None, :]).astype(jnp.int32), axis=1)
    overflow = te >= E
    te = jnp.where(overflow, E - 1, te).astype(jnp.int32)
    cum_excl = cum - nt
    off = jnp.where(overflow, jnp.maximum(nt[E - 1] - 1, 0),
                    i40 - cum_excl[te]).astype(jnp.int32)
    valid = (~overflow).astype(jnp.int32)

    d0 = didx[:, 0]
    d1 = didx[:, 1]
    g0 = gate[:, 0]
    g1 = gate[:, 1]
    d0w = d0.reshape(NW, TW)
    d1w = d1.reshape(NW, TW)
    d0c = d0.reshape(NW * 2, CW)
    d1c = d1.reshape(NW * 2, CW)
    g0rep = jnp.broadcast_to(g0[:, None], (T, GREP))
    g1rep = jnp.broadcast_to(g1[:, None], (T, GREP))

    xs, gs = _sc_dispatch(x2, d0w, d1w, g0rep, g1rep)

    grid_spec = pltpu.PrefetchScalarGridSpec(
        num_scalar_prefetch=3,
        grid=(NTILE,),
        in_specs=[
            pl.BlockSpec((G, D), lambda i, t, o, v: (t[i] * TPG + o[i], 0)),
            pl.BlockSpec((G, GREP), lambda i, t, o, v: (t[i] * TPG + o[i], 0)),
            pl.BlockSpec((1, H, D), lambda i, t, o, v: (t[i], 0, 0)),
            pl.BlockSpec((1, D, H), lambda i, t, o, v: (t[i], 0, 0)),
        ],
        out_specs=pl.BlockSpec((G, D),
                               lambda i, t, o, v: (t[i] * TPG + o[i], 0)),
    )
    ys = pl.pallas_call(
        _gmm_body,
        grid_spec=grid_spec,
        out_shape=jax.ShapeDtypeStruct((NSLOT, D), jnp.float32),
    )(te, off, valid, xs, gs, W1, W2)

    out2 = _sc_combine(ys, d0c, d1c)
    return out2.reshape(1, T, D)


# speedup vs baseline: 1.3312x; 1.0077x over previous
import functools

import jax
import jax.numpy as jnp
from jax import lax
from jax.experimental import pallas as pl
from jax.experimental.pallas import tpu as pltpu
from jax.experimental.pallas import tpu_sc as plsc

D = 1024
H = 2048
E = 8
FLAGS_DIM = 8
T = 2048
BLOCK = 128

CAP = 2048
NSLOT = E * CAP
G = 256
TPG = CAP // G
NTILE = (2 * T) // G + E
NW = 32
TW = T // NW
GREP = 128


def _router_body(x_ref, f_ref, rwx_ref, rwe_ref, rwf_ref, rb_ref,
                 pinv_ref, pscale_ref, pcarry_ref,
                 didx_ref, gate_ref, counts_ref, carry_ref, run_ref):
    i = pl.program_id(0)

    @pl.when(i == 0)
    def _():
        carry_ref[...] = jnp.zeros_like(carry_ref)
        run_ref[...] = jnp.zeros_like(run_ref)

    xb = x_ref[...]
    u = xb * pinv_ref[:, :1]
    p = u
    for k in (1, 2, 4, 8, 16, 32, 64):
        shifted = jnp.concatenate(
            [jnp.zeros((k, D), jnp.float32), p[:BLOCK - k, :]], axis=0)
        p = p + shifted
    yb = pcarry_ref[:, :1] * carry_ref[0:1, :] + pscale_ref[:, :1] * p
    yb = jnp.where(jnp.isfinite(yb), yb, 0.0)
    carry_ref[0:1, :] = yb[BLOCK - 1:BLOCK, :]

    fb = f_ref[...]
    logits = (
        jax.lax.dot_general(xb, rwx_ref[...], (((1,), (0,)), ((), ())),
                            preferred_element_type=jnp.float32)
        + jax.lax.dot_general(yb, rwe_ref[...], (((1,), (0,)), ((), ())),
                              preferred_element_type=jnp.float32)
        + jax.lax.dot_general(fb, rwf_ref[...], (((1,), (0,)), ((), ())),
                              preferred_element_type=jnp.float32)
        + rb_ref[0:1, :]
    )

    lane = jax.lax.broadcasted_iota(jnp.int32, (BLOCK, E), 1)
    m1 = jnp.max(logits, axis=1, keepdims=True)
    idx1 = jnp.min(jnp.where(logits == m1, lane, E), axis=1, keepdims=True)
    sel1 = lane == idx1
    masked = jnp.where(sel1, -jnp.inf, logits)
    m2 = jnp.max(masked, axis=1, keepdims=True)
    idx2 = jnp.min(jnp.where(masked == m2, lane, E), axis=1, keepdims=True)
    sel2 = lane == idx2
    ex = jnp.exp(m2 - m1)
    s = 1.0 + ex
    g1 = 1.0 / s
    g2 = ex / s

    blk = sel1.astype(jnp.int32) + sel2.astype(jnp.int32)
    pc = blk
    for k in (1, 2, 4, 8, 16, 32, 64):
        shifted = jnp.concatenate(
            [jnp.zeros((k, E), jnp.int32), pc[:BLOCK - k, :]], axis=0)
        pc = pc + shifted
    excl = pc - blk
    runrow = run_ref[0:1, :]
    base = excl + runrow + lane * CAP
    d0 = jnp.sum(jnp.where(sel1, base, 0), axis=1, keepdims=True)
    d1 = jnp.sum(jnp.where(sel2, base, 0), axis=1, keepdims=True)
    run_ref[0:1, :] = runrow + pc[BLOCK - 1:BLOCK, :]
    counts_ref[...] = jnp.broadcast_to(run_ref[0:1, :], (8, E))

    didx_ref[...] = jnp.where(lane == 0, d0, 0) + jnp.where(lane == 1, d1, 0)
    gate_ref[...] = jnp.where(lane == 0, g1, 0.0) + jnp.where(lane == 1, g2, 0.0)


_vector_mesh = plsc.VectorSubcoreMesh(core_axis_name="c", subcore_axis_name="s")


@functools.partial(
    pl.kernel,
    out_type=[jax.ShapeDtypeStruct((NSLOT, D), jnp.float32),
              jax.ShapeDtypeStruct((NSLOT, GREP), jnp.float32)],
    mesh=_vector_mesh,
    scratch_types=[pltpu.VMEM((TW,), jnp.int32),
                   pltpu.VMEM((TW,), jnp.int32),
                   pltpu.VMEM((TW, D), jnp.float32),
                   pltpu.VMEM((TW, GREP), jnp.float32)],
)
def _sc_dispatch(x_hbm, d0_hbm, d1_hbm, g0_hbm, g1_hbm,
                 xs_hbm, gs_hbm, i0_v, i1_v, xrows_v, grows_v):
    wid = lax.axis_index("s") * 2 + lax.axis_index("c")
    base = wid * TW
    pltpu.sync_copy(d0_hbm.at[wid], i0_v)
    pltpu.sync_copy(d1_hbm.at[wid], i1_v)
    pltpu.sync_copy(x_hbm.at[pl.ds(base, TW)], xrows_v)
    pltpu.sync_copy(xrows_v, xs_hbm.at[i0_v])
    pltpu.sync_copy(xrows_v, xs_hbm.at[i1_v])
    pltpu.sync_copy(g0_hbm.at[pl.ds(base, TW)], grows_v)
    pltpu.sync_copy(grows_v, gs_hbm.at[i0_v])
    pltpu.sync_copy(g1_hbm.at[pl.ds(base, TW)], grows_v)
    pltpu.sync_copy(grows_v, gs_hbm.at[i1_v])


def _gmm_body(te_ref, off_ref, valid_ref, xs_ref, gs_ref, w1_ref, w2_ref,
              ys_ref):
    i = pl.program_id(0)

    @pl.when(valid_ref[i] != 0)
    def _():
        h = jax.lax.dot_general(xs_ref[...], w1_ref[0],
                                (((1,), (1,)), ((), ())),
                                preferred_element_type=jnp.float32)
        h = jnp.maximum(h, 0.0)
        y = jax.lax.dot_general(h, w2_ref[0], (((1,), (1,)), ((), ())),
                                preferred_element_type=jnp.float32)
        ys_ref[...] = y * gs_ref[:, 0:1]


CW = TW // 2


@functools.partial(
    pl.kernel,
    out_type=jax.ShapeDtypeStruct((T, D), jnp.float32),
    mesh=_vector_mesh,
    scratch_types=[pltpu.VMEM((CW,), jnp.int32),
                   pltpu.VMEM((CW,), jnp.int32),
                   pltpu.VMEM((CW, D), jnp.float32),
                   pltpu.VMEM((CW, D), jnp.float32)],
)
def _sc_combine(ys_hbm, d0_hbm, d1_hbm, out_hbm, i0_v, i1_v, b0_v, b1_v):
    wid = lax.axis_index("s") * 2 + lax.axis_index("c")
    for ci in range(2):
        row = wid * 2 + ci
        pltpu.sync_copy(d0_hbm.at[row], i0_v)
        pltpu.sync_copy(d1_hbm.at[row], i1_v)
        pltpu.sync_copy(ys_hbm.at[i0_v], b0_v)
        pltpu.sync_copy(ys_hbm.at[i1_v], b1_v)

        @pl.loop(0, CW)
        def _(r):
            b0_v[r, :] = b0_v[r, :] + b1_v[r, :]

        pltpu.sync_copy(b0_v, out_hbm.at[pl.ds(row * CW, CW)])


def kernel(x, flags, W1, W2, router_w, router_b, alpha, layer_idx):
    x2 = x.reshape(T, D)
    flags2 = flags.reshape(T, FLAGS_DIM)
    rw = jax.lax.dynamic_index_in_dim(router_w, layer_idx, 0, keepdims=False)
    rb = jax.lax.dynamic_index_in_dim(router_b, layer_idx, 0, keepdims=False)
    a = jnp.clip(
        jnp.nan_to_num(alpha[layer_idx].astype(jnp.float32),
                       nan=0.8, posinf=0.8, neginf=0.8),
        1e-4, 0.9999)
    ar = jnp.arange(BLOCK, dtype=jnp.float32)
    pow_a = a ** ar
    pow_ap1 = a ** (ar + 1.0)
    pow_inv = a ** (-ar)
    one_minus = 1.0 - a
    P_inv = jnp.broadcast_to(pow_inv[:, None], (BLOCK, BLOCK))
    P_scale = jnp.broadcast_to((one_minus * pow_a)[:, None], (BLOCK, BLOCK))
    P_carry = jnp.broadcast_to(pow_ap1[:, None], (BLOCK, BLOCK))
    rwxT = rw[:, :D].T
    rweT = rw[:, D:2 * D].T
    rwfT = rw[:, 2 * D:].T
    rb8 = jnp.broadcast_to(rb[None, :], (8, E))

    didx, gate, counts8 = pl.pallas_call(
        _router_body,
        grid=(T // BLOCK,),
        in_specs=[
            pl.BlockSpec((BLOCK, D), lambda i: (i, 0)),
            pl.BlockSpec((BLOCK, FLAGS_DIM), lambda i: (i, 0)),
            pl.BlockSpec((D, E), lambda i: (0, 0)),
            pl.BlockSpec((D, E), lambda i: (0, 0)),
            pl.BlockSpec((FLAGS_DIM, E), lambda i: (0, 0)),
            pl.BlockSpec((8, E), lambda i: (0, 0)),
            pl.BlockSpec((BLOCK, BLOCK), lambda i: (0, 0)),
            pl.BlockSpec((BLOCK, BLOCK), lambda i: (0, 0)),
            pl.BlockSpec((BLOCK, BLOCK), lambda i: (0, 0)),
        ],
        out_specs=[
            pl.BlockSpec((BLOCK, E), lambda i: (i, 0)),
            pl.BlockSpec((BLOCK, E), lambda i: (i, 0)),
            pl.BlockSpec((8, E), lambda i: (0, 0)),
        ],
        out_shape=[
            jax.ShapeDtypeStruct((T, E), jnp.int32),
            jax.ShapeDtypeStruct((T, E), jnp.float32),
            jax.ShapeDtypeStruct((8, E), jnp.int32),
        ],
        scratch_shapes=[pltpu.VMEM((8, D), jnp.float32),
                        pltpu.VMEM((8, E), jnp.int32)],
    )(x2, flags2, rwxT, rweT, rwfT, rb8, P_inv, P_scale, P_carry)

    counts = counts8[0]
    nt = (counts + (G - 1)) // G
    cum = jnp.cumsum(nt)
    i40 = jnp.arange(NTILE, dtype=jnp.int32)
    te = jnp.sum((i40[:, None] >= cum[None, :]).astype(jnp.int32), axis=1)
    overflow = te >= E
    te = jnp.where(overflow, E - 1, te).astype(jnp.int32)
    cum_excl = cum - nt
    off = jnp.where(overflow, jnp.maximum(nt[E - 1] - 1, 0),
                    i40 - cum_excl[te]).astype(jnp.int32)
    valid = (~overflow).astype(jnp.int32)

    d0 = didx[:, 0]
    d1 = didx[:, 1]
    g0 = gate[:, 0]
    g1 = gate[:, 1]
    d0w = d0.reshape(NW, TW)
    d1w = d1.reshape(NW, TW)
    d0c = d0.reshape(NW * 2, CW)
    d1c = d1.reshape(NW * 2, CW)
    g0rep = jnp.broadcast_to(g0[:, None], (T, GREP))
    g1rep = jnp.broadcast_to(g1[:, None], (T, GREP))

    xs, gs = _sc_dispatch(x2, d0w, d1w, g0rep, g1rep)

    grid_spec = pltpu.PrefetchScalarGridSpec(
        num_scalar_prefetch=3,
        grid=(NTILE,),
        in_specs=[
            pl.BlockSpec((G, D), lambda i, t, o, v: (t[i] * TPG + o[i], 0)),
            pl.BlockSpec((G, GREP), lambda i, t, o, v: (t[i] * TPG + o[i], 0)),
            pl.BlockSpec((1, H, D), lambda i, t, o, v: (t[i], 0, 0)),
            pl.BlockSpec((1, D, H), lambda i, t, o, v: (t[i], 0, 0)),
        ],
        out_specs=pl.BlockSpec((G, D),
                               lambda i, t, o, v: (t[i] * TPG + o[i], 0)),
    )
    ys = pl.pallas_call(
        _gmm_body,
        grid_spec=grid_spec,
        out_shape=jax.ShapeDtypeStruct((NSLOT, D), jnp.float32),
    )(te, off, valid, xs, gs, W1, W2)

    out2 = _sc_combine(ys, d0c, d1c)
    return out2.reshape(1, T, D)


# speedup vs baseline: 1.3602x; 1.0218x over previous
import functools

import jax
import jax.numpy as jnp
from jax import lax
from jax.experimental import pallas as pl
from jax.experimental.pallas import tpu as pltpu
from jax.experimental.pallas import tpu_sc as plsc

D = 1024
H = 2048
E = 8
FLAGS_DIM = 8
T = 2048
BLOCK = 128

CAP = 2048
NSLOT = E * CAP
G = 256
TPG = CAP // G
NTILE = (2 * T) // G + E
NW = 32
TW = T // NW
GREP = 128


def _router_body(x_ref, f_ref, rwx_ref, rwe_ref, rwf_ref, rb_ref,
                 ema_ref, pcarry_ref,
                 didx_ref, gate_ref, counts_ref, carry_ref, run_ref):
    i = pl.program_id(0)

    @pl.when(i == 0)
    def _():
        carry_ref[...] = jnp.zeros_like(carry_ref)
        run_ref[...] = jnp.zeros_like(run_ref)

    xb = x_ref[...]
    yb = (jax.lax.dot_general(ema_ref[...], xb, (((1,), (0,)), ((), ())),
                              preferred_element_type=jnp.float32)
          + pcarry_ref[:, :1] * carry_ref[0:1, :])
    yb = jnp.where(jnp.isfinite(yb), yb, 0.0)
    carry_ref[0:1, :] = yb[BLOCK - 1:BLOCK, :]

    fb = f_ref[...]
    logits = (
        jax.lax.dot_general(xb, rwx_ref[...], (((1,), (0,)), ((), ())),
                            preferred_element_type=jnp.float32)
        + jax.lax.dot_general(yb, rwe_ref[...], (((1,), (0,)), ((), ())),
                              preferred_element_type=jnp.float32)
        + jax.lax.dot_general(fb, rwf_ref[...], (((1,), (0,)), ((), ())),
                              preferred_element_type=jnp.float32)
        + rb_ref[0:1, :]
    )

    lane = jax.lax.broadcasted_iota(jnp.int32, (BLOCK, E), 1)
    m1 = jnp.max(logits, axis=1, keepdims=True)
    idx1 = jnp.min(jnp.where(logits == m1, lane, E), axis=1, keepdims=True)
    sel1 = lane == idx1
    masked = jnp.where(sel1, -jnp.inf, logits)
    m2 = jnp.max(masked, axis=1, keepdims=True)
    idx2 = jnp.min(jnp.where(masked == m2, lane, E), axis=1, keepdims=True)
    sel2 = lane == idx2
    ex = jnp.exp(m2 - m1)
    s = 1.0 + ex
    g1 = 1.0 / s
    g2 = ex / s

    blk = sel1.astype(jnp.int32) + sel2.astype(jnp.int32)
    pc = blk
    for k in (1, 2, 4, 8, 16, 32, 64):
        shifted = jnp.concatenate(
            [jnp.zeros((k, E), jnp.int32), pc[:BLOCK - k, :]], axis=0)
        pc = pc + shifted
    excl = pc - blk
    runrow = run_ref[0:1, :]
    base = excl + runrow + lane * CAP
    d0 = jnp.sum(jnp.where(sel1, base, 0), axis=1, keepdims=True)
    d1 = jnp.sum(jnp.where(sel2, base, 0), axis=1, keepdims=True)
    run_ref[0:1, :] = runrow + pc[BLOCK - 1:BLOCK, :]
    counts_ref[...] = jnp.broadcast_to(run_ref[0:1, :], (8, E))

    didx_ref[...] = jnp.where(lane == 0, d0, 0) + jnp.where(lane == 1, d1, 0)
    gate_ref[...] = jnp.where(lane == 0, g1, 0.0) + jnp.where(lane == 1, g2, 0.0)


_vector_mesh = plsc.VectorSubcoreMesh(core_axis_name="c", subcore_axis_name="s")


@functools.partial(
    pl.kernel,
    out_type=[jax.ShapeDtypeStruct((NSLOT, D), jnp.float32),
              jax.ShapeDtypeStruct((NSLOT, GREP), jnp.float32)],
    mesh=_vector_mesh,
    scratch_types=[pltpu.VMEM((TW,), jnp.int32),
                   pltpu.VMEM((TW,), jnp.int32),
                   pltpu.VMEM((TW, D), jnp.float32),
                   pltpu.VMEM((TW, GREP), jnp.float32)],
)
def _sc_dispatch(x_hbm, d0_hbm, d1_hbm, g0_hbm, g1_hbm,
                 xs_hbm, gs_hbm, i0_v, i1_v, xrows_v, grows_v):
    wid = lax.axis_index("s") * 2 + lax.axis_index("c")
    base = wid * TW
    pltpu.sync_copy(d0_hbm.at[wid], i0_v)
    pltpu.sync_copy(d1_hbm.at[wid], i1_v)
    pltpu.sync_copy(x_hbm.at[pl.ds(base, TW)], xrows_v)
    pltpu.sync_copy(xrows_v, xs_hbm.at[i0_v])
    pltpu.sync_copy(xrows_v, xs_hbm.at[i1_v])
    pltpu.sync_copy(g0_hbm.at[pl.ds(base, TW)], grows_v)
    pltpu.sync_copy(grows_v, gs_hbm.at[i0_v])
    pltpu.sync_copy(g1_hbm.at[pl.ds(base, TW)], grows_v)
    pltpu.sync_copy(grows_v, gs_hbm.at[i1_v])


def _gmm_body(te_ref, off_ref, valid_ref, xs_ref, gs_ref, w1_ref, w2_ref,
              ys_ref):
    i = pl.program_id(0)

    @pl.when(valid_ref[i] != 0)
    def _():
        h = jax.lax.dot_general(xs_ref[...], w1_ref[0],
                                (((1,), (1,)), ((), ())),
                                preferred_element_type=jnp.float32)
        h = jnp.maximum(h, 0.0)
        y = jax.lax.dot_general(h, w2_ref[0], (((1,), (1,)), ((), ())),
                                preferred_element_type=jnp.float32)
        ys_ref[...] = y * gs_ref[:, 0:1]


CW = TW // 2


@functools.partial(
    pl.kernel,
    out_type=jax.ShapeDtypeStruct((T, D), jnp.float32),
    mesh=_vector_mesh,
    scratch_types=[pltpu.VMEM((CW,), jnp.int32),
                   pltpu.VMEM((CW,), jnp.int32),
                   pltpu.VMEM((CW, D), jnp.float32),
                   pltpu.VMEM((CW, D), jnp.float32)],
)
def _sc_combine(ys_hbm, d0_hbm, d1_hbm, out_hbm, i0_v, i1_v, b0_v, b1_v):
    wid = lax.axis_index("s") * 2 + lax.axis_index("c")
    for ci in range(2):
        row = wid * 2 + ci
        pltpu.sync_copy(d0_hbm.at[row], i0_v)
        pltpu.sync_copy(d1_hbm.at[row], i1_v)
        pltpu.sync_copy(ys_hbm.at[i0_v], b0_v)
        pltpu.sync_copy(ys_hbm.at[i1_v], b1_v)

        @pl.loop(0, CW)
        def _(r):
            b0_v[r, :] = b0_v[r, :] + b1_v[r, :]

        pltpu.sync_copy(b0_v, out_hbm.at[pl.ds(row * CW, CW)])


def kernel(x, flags, W1, W2, router_w, router_b, alpha, layer_idx):
    x2 = x.reshape(T, D)
    flags2 = flags.reshape(T, FLAGS_DIM)
    rw = jax.lax.dynamic_index_in_dim(router_w, layer_idx, 0, keepdims=False)
    rb = jax.lax.dynamic_index_in_dim(router_b, layer_idx, 0, keepdims=False)
    a = jnp.clip(
        jnp.nan_to_num(alpha[layer_idx].astype(jnp.float32),
                       nan=0.8, posinf=0.8, neginf=0.8),
        1e-4, 0.9999)
    ar = jnp.arange(BLOCK, dtype=jnp.float32)
    pow_ap1 = a ** (ar + 1.0)
    one_minus = 1.0 - a
    dd = ar[:, None] - ar[None, :]
    A_ema = jnp.where(dd >= 0.0, one_minus * a ** jnp.maximum(dd, 0.0), 0.0)
    P_carry = jnp.broadcast_to(pow_ap1[:, None], (BLOCK, BLOCK))
    rwxT = rw[:, :D].T
    rweT = rw[:, D:2 * D].T
    rwfT = rw[:, 2 * D:].T
    rb8 = jnp.broadcast_to(rb[None, :], (8, E))

    didx, gate, counts8 = pl.pallas_call(
        _router_body,
        grid=(T // BLOCK,),
        in_specs=[
            pl.BlockSpec((BLOCK, D), lambda i: (i, 0)),
            pl.BlockSpec((BLOCK, FLAGS_DIM), lambda i: (i, 0)),
            pl.BlockSpec((D, E), lambda i: (0, 0)),
            pl.BlockSpec((D, E), lambda i: (0, 0)),
            pl.BlockSpec((FLAGS_DIM, E), lambda i: (0, 0)),
            pl.BlockSpec((8, E), lambda i: (0, 0)),
            pl.BlockSpec((BLOCK, BLOCK), lambda i: (0, 0)),
            pl.BlockSpec((BLOCK, BLOCK), lambda i: (0, 0)),
        ],
        out_specs=[
            pl.BlockSpec((BLOCK, E), lambda i: (i, 0)),
            pl.BlockSpec((BLOCK, E), lambda i: (i, 0)),
            pl.BlockSpec((8, E), lambda i: (0, 0)),
        ],
        out_shape=[
            jax.ShapeDtypeStruct((T, E), jnp.int32),
            jax.ShapeDtypeStruct((T, E), jnp.float32),
            jax.ShapeDtypeStruct((8, E), jnp.int32),
        ],
        scratch_shapes=[pltpu.VMEM((8, D), jnp.float32),
                        pltpu.VMEM((8, E), jnp.int32)],
    )(x2, flags2, rwxT, rweT, rwfT, rb8, A_ema, P_carry)

    counts = counts8[0]
    nt = (counts + (G - 1)) // G
    cum = jnp.cumsum(nt)
    i40 = jnp.arange(NTILE, dtype=jnp.int32)
    te = jnp.sum((i40[:, None] >= cum[None, :]).astype(jnp.int32), axis=1)
    overflow = te >= E
    te = jnp.where(overflow, E - 1, te).astype(jnp.int32)
    cum_excl = cum - nt
    off = jnp.where(overflow, jnp.maximum(nt[E - 1] - 1, 0),
                    i40 - cum_excl[te]).astype(jnp.int32)
    valid = (~overflow).astype(jnp.int32)

    d0 = didx[:, 0]
    d1 = didx[:, 1]
    g0 = gate[:, 0]
    g1 = gate[:, 1]
    d0w = d0.reshape(NW, TW)
    d1w = d1.reshape(NW, TW)
    d0c = d0.reshape(NW * 2, CW)
    d1c = d1.reshape(NW * 2, CW)
    g0rep = jnp.broadcast_to(g0[:, None], (T, GREP))
    g1rep = jnp.broadcast_to(g1[:, None], (T, GREP))

    xs, gs = _sc_dispatch(x2, d0w, d1w, g0rep, g1rep)

    grid_spec = pltpu.PrefetchScalarGridSpec(
        num_scalar_prefetch=3,
        grid=(NTILE,),
        in_specs=[
            pl.BlockSpec((G, D), lambda i, t, o, v: (t[i] * TPG + o[i], 0)),
            pl.BlockSpec((G, GREP), lambda i, t, o, v: (t[i] * TPG + o[i], 0)),
            pl.BlockSpec((1, H, D), lambda i, t, o, v: (t[i], 0, 0)),
            pl.BlockSpec((1, D, H), lambda i, t, o, v: (t[i], 0, 0)),
        ],
        out_specs=pl.BlockSpec((G, D),
                               lambda i, t, o, v: (t[i] * TPG + o[i], 0)),
    )
    ys = pl.pallas_call(
        _gmm_body,
        grid_spec=grid_spec,
        out_shape=jax.ShapeDtypeStruct((NSLOT, D), jnp.float32),
    )(te, off, valid, xs, gs, W1, W2)

    out2 = _sc_combine(ys, d0c, d1c)
    return out2.reshape(1, T, D)
